# Initial kernel scaffold; baseline (speedup 1.0000x reference)
import jax
import jax.numpy as jnp
from jax.experimental import pallas as pl


def kernel(x, edge_index, edge_attr, W1, as1, ad1, We1, ae1, b1, W2, as2, ad2, We2, ae2, b2):
    raise NotImplementedError("write your pallas kernel here")


# speedup vs baseline: 1.0765x; 1.0765x over previous
import jax
import jax.numpy as jnp
from jax.experimental import pallas as pl


def _mm_body(x_ref, w_ref, o_ref):
    o_ref[...] = x_ref[...] @ w_ref[...]


def _matmul(x, w):
    n, k = x.shape
    _, d = w.shape
    blk = 1000
    return pl.pallas_call(
        _mm_body,
        grid=(n // blk,),
        in_specs=[
            pl.BlockSpec((blk, k), lambda i: (i, 0)),
            pl.BlockSpec((k, d), lambda i: (0, 0)),
        ],
        out_specs=pl.BlockSpec((blk, d), lambda i: (i, 0)),
        out_shape=jax.ShapeDtypeStruct((n, d), x.dtype),
    )(x, w)


def _gat_layer(x, src, dst, ea, W, a_s, a_d, b):
    n = x.shape[0]
    h = _matmul(x, W)
    alpha = (h @ a_s)[src] + (h @ a_d)[dst] + ea
    alpha = jax.nn.leaky_relu(alpha, negative_slope=0.2)
    m = jax.ops.segment_max(alpha, dst, num_segments=n)
    m = jnp.where(jnp.isfinite(m), m, 0.0)
    ex = jnp.exp(alpha - m[dst])
    s = jax.ops.segment_sum(ex, dst, num_segments=n)
    alpha = ex / (s[dst] + 1e-16)
    msg = h[src] * alpha[:, None]
    return jax.ops.segment_sum(msg, dst, num_segments=n) + b


def kernel(x, edge_index, edge_attr, W1, as1, ad1, We1, ae1, b1, W2, as2, ad2, We2, ae2, b2):
    src = edge_index[0]
    dst = edge_index[1]
    ea1 = edge_attr @ (We1 @ ae1)
    ea2 = edge_attr @ (We2 @ ae2)
    h = _gat_layer(x, src, dst, ea1, W1, as1, ad1, b1)
    h = jax.nn.relu(h)
    return _gat_layer(h, src, dst, ea2, W2, as2, ad2, b2)
